# Initial kernel scaffold; baseline (speedup 1.0000x reference)
import jax
import jax.numpy as jnp
from jax.experimental import pallas as pl


def kernel(query_coords, query_features, key_coords, key_features):
    raise NotImplementedError("write your pallas kernel here")


# speedup vs baseline: 3.4741x; 3.4741x over previous
import functools

import jax
import jax.numpy as jnp
from jax import lax
from jax.experimental import pallas as pl
from jax.experimental.pallas import tpu as pltpu
from jax.experimental.pallas import tpu_sc as plsc

KNN = 8
B = 4
NQ = 2048
NK = 2048
C = 256
QB = 256


def _topk_body(qc_ref, kc_ref, idx_ref):
    qc = qc_ref[0]
    kc = kc_ref[0]
    s = lax.dot_general(qc, kc, (((0,), (0,)), ((), ())),
                        preferred_element_type=jnp.float32)
    qq = jnp.sum(qc * qc, axis=0)
    kk = jnp.sum(kc * kc, axis=0)
    d = (qq[:, None] - 2.0 * s) + kk[None, :]
    col = lax.broadcasted_iota(jnp.int32, (QB, NK), 1)
    kcol = lax.broadcasted_iota(jnp.int32, (QB, KNN), 1)
    acc = jnp.zeros((QB, KNN), jnp.int32)
    for j in range(KNN):
        m = jnp.min(d, axis=1, keepdims=True)
        amin = jnp.min(jnp.where(d == m, col, NK), axis=1, keepdims=True)
        acc = jnp.where(kcol == j, amin, acc)
        d = jnp.where(col == amin, jnp.inf, d)
    idx_ref[0] = acc


def _knn_idx(query_coords, key_coords):
    return pl.pallas_call(
        _topk_body,
        grid=(B, NQ // QB),
        in_specs=[
            pl.BlockSpec((1, 3, QB), lambda b, q: (b, 0, q)),
            pl.BlockSpec((1, 3, NK), lambda b, q: (b, 0, 0)),
        ],
        out_specs=pl.BlockSpec((1, QB, KNN), lambda b, q: (b, q, 0)),
        out_shape=jax.ShapeDtypeStruct((B, NQ, KNN), jnp.int32),
        compiler_params=pltpu.CompilerParams(
            dimension_semantics=("parallel", "parallel")),
    )(query_coords, key_coords)


@functools.lru_cache(maxsize=1)
def _make_sc_gather():
    mesh = plsc.VectorSubcoreMesh(core_axis_name="c", subcore_axis_name="s")
    n_vec = NQ * KNN // 16

    @functools.partial(
        pl.kernel,
        mesh=mesh,
        out_type=jax.ShapeDtypeStruct((B * 2 * C, NQ * KNN), jnp.float32),
        scratch_types=[
            pltpu.VMEM((NQ * KNN,), jnp.int32),
            pltpu.VMEM((NQ * KNN,), jnp.int32),
            pltpu.VMEM((NK,), jnp.float32),
            pltpu.VMEM((NQ,), jnp.float32),
            pltpu.VMEM((NQ * KNN,), jnp.float32),
            pltpu.VMEM((NQ * KNN,), jnp.float32),
        ],
        compiler_params=pltpu.CompilerParams(needs_layout_passes=False),
    )
    def sc_gather(idx_hbm, kf_hbm, qf_hbm, out_hbm,
                  scr_v, scrp_v, krow_v, qrow_v, o1_v, o2_v):
        cid = lax.axis_index("c")
        sid = lax.axis_index("s")
        wid = sid * 2 + cid
        b = wid // 8
        slot = wid % 8
        lane = lax.iota(jnp.int32, 16)
        offs = (lane % 8) * NQ + (lane // 8)
        qpos0 = lane // 8

        pltpu.sync_copy(idx_hbm.at[b], scr_v)

        def build(j, carry):
            v = plsc.load_gather(scr_v, [offs + 2 * j])
            scrp_v[pl.ds(pl.multiple_of(j * 16, 16), 16)] = v
            return carry
        lax.fori_loop(0, n_vec, build, 0)

        def chan(i, carry):
            gc = b * C + slot * 32 + i
            pltpu.sync_copy(kf_hbm.at[gc], krow_v)
            pltpu.sync_copy(qf_hbm.at[gc], qrow_v)

            def inner(j, c2):
                base = pl.multiple_of(j * 16, 16)
                idxv = scrp_v[pl.ds(base, 16)]
                sv = plsc.load_gather(krow_v, [idxv])
                qv = plsc.load_gather(qrow_v, [qpos0 + 2 * j])
                o1_v[pl.ds(base, 16)] = sv - qv
                o2_v[pl.ds(base, 16)] = qv
                return c2
            lax.fori_loop(0, n_vec, inner, 0)

            r1 = b * (2 * C) + slot * 32 + i
            pltpu.sync_copy(o1_v, out_hbm.at[r1])
            pltpu.sync_copy(o2_v, out_hbm.at[r1 + C])
            return carry
        lax.fori_loop(0, C // 8, chan, 0)

    return sc_gather


def kernel(query_coords, query_features, key_coords, key_features):
    idx = _knn_idx(query_coords, key_coords)
    idx2 = idx.reshape(B, NQ * KNN)
    kf2 = key_features.reshape(B * C, NK)
    qf2 = query_features.reshape(B * C, NQ)
    out = _make_sc_gather()(idx2, kf2, qf2)
    return out.reshape(B, 2 * C, NQ, KNN)
